# baseline (speedup 1.0000x reference)
import jax
import jax.numpy as jnp
from jax.experimental import pallas as pl


def _fused_sage_kernel2(a0_ref, a1_ref, x_ref, w_ref, b_ref, out_ref):
    f = x_ref.shape[1]
    bm = a0_ref.shape[0]
    i = pl.program_id(0)
    w1 = w_ref[:f, :]
    w2 = w_ref[f:, :]
    for k, a_ref in enumerate((a0_ref, a1_ref)):
        xb = x_ref[pl.ds((2 * i + k) * bm, bm), :]
        agg = jnp.dot(a_ref[...], x_ref[...], preferred_element_type=jnp.float32)
        out = jnp.dot(xb, w1, preferred_element_type=jnp.float32)
        out += jnp.dot(agg, w2, preferred_element_type=jnp.float32)
        out += b_ref[...]
        out_ref[k * bm:(k + 1) * bm, :] = jnp.maximum(out, 0.0)


def kernel(x, norm_GraphSAGE, W, b):
    n, f = x.shape
    f_out = W.shape[1]
    bm = 200
    steps = n // (2 * bm)
    assert n % (2 * bm) == 0
    b2 = b.reshape(1, f_out)
    return pl.pallas_call(
        _fused_sage_kernel2,
        grid=(steps,),
        in_specs=[
            pl.BlockSpec((bm, n), lambda i: (2 * i, 0)),
            pl.BlockSpec((bm, n), lambda i: (2 * i + 1, 0)),
            pl.BlockSpec((n, f), lambda i: (0, 0)),
            pl.BlockSpec(W.shape, lambda i: (0, 0)),
            pl.BlockSpec((1, f_out), lambda i: (0, 0)),
        ],
        out_specs=pl.BlockSpec((2 * bm, f_out), lambda i: (i, 0)),
        out_shape=jax.ShapeDtypeStruct((n, f_out), jnp.float32),
    )(norm_GraphSAGE, norm_GraphSAGE, x, W, b2)
